# Initial kernel scaffold; baseline (speedup 1.0000x reference)
import jax
import jax.numpy as jnp
from jax.experimental import pallas as pl


def kernel(x, norm1_scale, norm2_scale, qw, qb, kw, kb, vw, vb, gw, gb, ow, ob, router_w, router_b, e_w1, e_b1, e_w2, e_b2):
    raise NotImplementedError("write your pallas kernel here")


# speedup vs baseline: 3.3245x; 3.3245x over previous
import jax
import jax.numpy as jnp
from jax.experimental import pallas as pl
from jax.experimental.pallas import tpu as pltpu

D = 512
H = 8
DK = 64
CHUNK = 64
E = 4
TOPK = 2
MULT = 4
CAPF = 1.25
LANES = 128

BT = 256
BTE = 512


def _attn_router_body(x_ref, n1_ref, n2_ref, wq_ref, wqp_ref, wk_ref, wkp_ref,
                      wv_ref, wg_ref, wo_ref, bias_ref, cos_ref, sin_ref,
                      rw_ref, rb_ref,
                      x2_ref, xn2_ref, gates_ref, stats_ref,
                      S_ref, Z_ref, cnt_ref, *, cap):
    b = pl.program_id(0)
    t = pl.program_id(1)

    @pl.when(t == 0)
    def _():
        S_ref[...] = jnp.zeros_like(S_ref)
        Z_ref[...] = jnp.zeros_like(Z_ref)

    @pl.when(jnp.logical_and(b == 0, t == 0))
    def _():
        cnt_ref[...] = jnp.zeros_like(cnt_ref)

    xb = x_ref[0]
    ms = jnp.mean(xb * xb, axis=-1, keepdims=True)
    xn = xb * jax.lax.rsqrt(ms) * n1_ref[...]

    q = xn @ wq_ref[...] + bias_ref[0:1, :]
    qp = xn @ wqp_ref[...] + bias_ref[1:2, :]
    k = xn @ wk_ref[...] + bias_ref[2:3, :]
    kp = xn @ wkp_ref[...] + bias_ref[3:4, :]
    v = xn @ wv_ref[...] + bias_ref[4:5, :]
    gp = xn @ wg_ref[...] + bias_ref[5:6, :]
    g = 1.0 / (1.0 + jnp.exp(-gp))

    cosb = cos_ref[...]
    sinb = sin_ref[...]
    qr = q * cosb + qp * sinb
    kr = k * cosb + kp * sinb
    qe = jnp.where(qr > 0, qr + 1.0, jnp.exp(qr))
    ke = jnp.where(kr > 0, kr + 1.0, jnp.exp(kr))

    ri = jax.lax.broadcasted_iota(jnp.int32, (D, D), 0) // DK
    ci = jax.lax.broadcasted_iota(jnp.int32, (D, D), 1) // DK
    bdf = (ri == ci).astype(jnp.float32)

    outs = []
    for c in range(BT // CHUNK):
        sl = slice(c * CHUNK, (c + 1) * CHUNK)
        qc, kc, vc, gc = qe[sl], ke[sl], v[sl], g[sl]
        m = jax.lax.dot_general(kc, vc, (((0,), (0,)), ((), ())),
                                preferred_element_type=jnp.float32)
        S_ref[...] += m * bdf
        Z_ref[0:1, :] += jnp.sum(kc, axis=0, keepdims=True)
        num = jax.lax.dot_general(qc, S_ref[...], (((1,), (0,)), ((), ())),
                                  preferred_element_type=jnp.float32)
        den = jax.lax.dot_general(qc * Z_ref[0:1, :], bdf,
                                  (((1,), (0,)), ((), ())),
                                  preferred_element_type=jnp.float32) + 1e-6
        outs.append(gc * num / den)
    attn = jnp.concatenate(outs, axis=0)
    x2 = xb + attn @ wo_ref[...] + bias_ref[6:7, :]
    x2_ref[0] = x2

    ms2 = jnp.mean(x2 * x2, axis=-1, keepdims=True)
    xn2 = x2 * jax.lax.rsqrt(ms2) * n2_ref[...]
    xn2_ref[...] = xn2

    logits = xn2 @ rw_ref[...] + rb_ref[...]
    mx = jnp.max(logits, axis=-1, keepdims=True)
    pe = jnp.exp(logits - mx)
    probs = pe / jnp.sum(pe, axis=-1, keepdims=True)
    cnt_ref[2:3, :] += jnp.sum(probs, axis=0, keepdims=True)

    lane = jax.lax.broadcasted_iota(jnp.int32, (BT, LANES), 1)
    m1 = jnp.max(probs, axis=-1, keepdims=True)
    i1 = jnp.min(jnp.where(probs == m1, lane, LANES), axis=-1, keepdims=True)
    pwo = jnp.where(lane == i1, -1.0, probs)
    m2 = jnp.max(pwo, axis=-1, keepdims=True)
    i2 = jnp.min(jnp.where(pwo == m2, lane, LANES), axis=-1, keepdims=True)
    tsum = m1 + m2
    tp1 = m1 / tsum
    tp2 = m2 / tsum

    oh1 = (lane == i1).astype(jnp.float32)
    oh2 = (lane == i2).astype(jnp.float32)
    mm = oh1 + oh2
    lt = (jax.lax.broadcasted_iota(jnp.int32, (BT, BT), 1) <
          jax.lax.broadcasted_iota(jnp.int32, (BT, BT), 0)).astype(jnp.float32)
    excl = jax.lax.dot_general(lt, mm, (((1,), (0,)), ((), ())),
                               preferred_element_type=jnp.float32)
    rank = cnt_ref[0:1, :] + excl
    keepm = (rank < float(cap)).astype(jnp.float32)
    keep1 = oh1 * keepm
    keep2 = oh2 * keepm
    gate = keep1 * tp1 + keep2 * tp2
    gates_ref[...] = gate
    cnt_ref[1:2, :] += jnp.sum(keep1 + keep2, axis=0, keepdims=True)
    cnt_ref[0:1, :] += jnp.sum(mm, axis=0, keepdims=True)
    stats_ref[...] = cnt_ref[...]


def _moe_dense_body(xn2_ref, x2_ref, gates_ref, w1_ref, b1_ref, w2_ref, b2_ref,
                    out_ref):
    e = pl.program_id(1)
    xn = xn2_ref[...]
    h = xn @ w1_ref[0] + b1_ref[0]
    h = 0.5 * h * (1.0 + jax.lax.erf(h * 0.7071067811865476))
    y = h @ w2_ref[0] + b2_ref[0]
    lane = jax.lax.broadcasted_iota(jnp.int32, (BTE, LANES), 1)
    ge = jnp.sum(jnp.where(lane == e, gates_ref[...], 0.0), axis=-1,
                 keepdims=True)

    @pl.when(e == 0)
    def _():
        out_ref[...] = x2_ref[...] + ge * y

    @pl.when(e != 0)
    def _():
        out_ref[...] += ge * y


def _swap_halves(w):
    shp = w.shape
    wr = w.reshape(shp[:-1] + (H, 2, DK // 2))
    return jnp.flip(wr, axis=-2).reshape(shp)


def kernel(x, norm1_scale, norm2_scale, qw, qb, kw, kb, vw, vb, gw, gb, ow, ob,
           router_w, router_b, e_w1, e_b1, e_w2, e_b2):
    B, T, _ = x.shape
    N = B * T
    NT = T // BT
    cap = int(CAPF * (N / E))
    hidden = D * MULT

    qwp = _swap_halves(qw)
    kwp = _swap_halves(kw)
    qbp = _swap_halves(qb)
    kbp = _swap_halves(kb)
    biasmat = jnp.stack([qb, qbp, kb, kbp, vb, gb, ob, jnp.zeros_like(ob)])

    half = DK // 2
    freqs = 1.0 / (10000.0 ** (jnp.arange(half, dtype=jnp.float32) / half))
    f = jnp.outer(jnp.arange(T, dtype=jnp.float32), freqs)
    cosb = jnp.cos(f)
    sinb = jnp.sin(f)
    cosF = jnp.tile(jnp.concatenate([cosb, cosb], axis=1), (1, H))
    sinF = jnp.tile(jnp.concatenate([-sinb, sinb], axis=1), (1, H))

    rw_pad = jnp.zeros((D, LANES), jnp.float32).at[:, :E].set(router_w)
    rb_pad = jnp.full((1, LANES), -1e30, jnp.float32).at[0, :E].set(router_b)

    import functools
    body = functools.partial(_attn_router_body, cap=cap)
    x2, xn2, gates, stats = pl.pallas_call(
        body,
        grid=(B, NT),
        in_specs=[
            pl.BlockSpec((1, BT, D), lambda b, t: (b, t, 0)),
            pl.BlockSpec((1, D), lambda b, t: (0, 0)),
            pl.BlockSpec((1, D), lambda b, t: (0, 0)),
            pl.BlockSpec((D, D), lambda b, t: (0, 0)),
            pl.BlockSpec((D, D), lambda b, t: (0, 0)),
            pl.BlockSpec((D, D), lambda b, t: (0, 0)),
            pl.BlockSpec((D, D), lambda b, t: (0, 0)),
            pl.BlockSpec((D, D), lambda b, t: (0, 0)),
            pl.BlockSpec((D, D), lambda b, t: (0, 0)),
            pl.BlockSpec((D, D), lambda b, t: (0, 0)),
            pl.BlockSpec((8, D), lambda b, t: (0, 0)),
            pl.BlockSpec((BT, D), lambda b, t: (t, 0)),
            pl.BlockSpec((BT, D), lambda b, t: (t, 0)),
            pl.BlockSpec((D, LANES), lambda b, t: (0, 0)),
            pl.BlockSpec((1, LANES), lambda b, t: (0, 0)),
        ],
        out_specs=[
            pl.BlockSpec((1, BT, D), lambda b, t: (b, t, 0)),
            pl.BlockSpec((BT, D), lambda b, t: (b * (T // BT) + t, 0)),
            pl.BlockSpec((BT, LANES), lambda b, t: (b * (T // BT) + t, 0)),
            pl.BlockSpec((8, LANES), lambda b, t: (0, 0)),
        ],
        out_shape=[
            jax.ShapeDtypeStruct((B, T, D), jnp.float32),
            jax.ShapeDtypeStruct((N, D), jnp.float32),
            jax.ShapeDtypeStruct((N, LANES), jnp.float32),
            jax.ShapeDtypeStruct((8, LANES), jnp.float32),
        ],
        scratch_shapes=[
            pltpu.VMEM((D, D), jnp.float32),
            pltpu.VMEM((8, D), jnp.float32),
            pltpu.VMEM((8, LANES), jnp.float32),
        ],
        compiler_params=pltpu.CompilerParams(
            dimension_semantics=("arbitrary", "arbitrary")),
    )(x, norm1_scale.reshape(1, D), norm2_scale.reshape(1, D),
      qw, qwp, kw, kwp, vw, gw, ow, biasmat, cosF, sinF, rw_pad, rb_pad)

    x2f = x2.reshape(N, D)
    out = pl.pallas_call(
        _moe_dense_body,
        grid=(N // BTE, E),
        in_specs=[
            pl.BlockSpec((BTE, D), lambda t, e: (t, 0)),
            pl.BlockSpec((BTE, D), lambda t, e: (t, 0)),
            pl.BlockSpec((BTE, LANES), lambda t, e: (t, 0)),
            pl.BlockSpec((1, D, hidden), lambda t, e: (e, 0, 0)),
            pl.BlockSpec((1, 1, hidden), lambda t, e: (e, 0, 0)),
            pl.BlockSpec((1, hidden, D), lambda t, e: (e, 0, 0)),
            pl.BlockSpec((1, 1, D), lambda t, e: (e, 0, 0)),
        ],
        out_specs=pl.BlockSpec((BTE, D), lambda t, e: (t, 0)),
        out_shape=jax.ShapeDtypeStruct((N, D), jnp.float32),
        compiler_params=pltpu.CompilerParams(
            dimension_semantics=("arbitrary", "arbitrary")),
    )(xn2, x2f, gates, e_w1, e_b1.reshape(E, 1, hidden), e_w2,
      e_b2.reshape(E, 1, D))

    imp = stats[2, :E]
    load = stats[1, :E]
    aux = jnp.sum((imp / jnp.sum(imp)) * (load / jnp.sum(load))) * float(E * E)
    return out.reshape(B, T, D), aux
